# baseline (speedup 1.0000x reference)
import functools

import jax
import jax.numpy as jnp
from jax import lax
from jax.experimental import pallas as pl
from jax.experimental.pallas import tpu as pltpu
from jax.experimental.pallas import tpu_sc as plsc

B, L = 4096, 200
NTOK = B * L
HTOK = NTOK // 2
EMB_DIM = 64
HID = 8

NC, NS = 2, 16
NW = NC * NS
TOK_PER_W = NTOK // NW
CHUNK = 1024
STREAM = 128
N_STEPS = TOK_PER_W // CHUNK
N_SUB = CHUNK // STREAM


def _sc_gather_body(table_hbm, idx_hbm, out_hbm, idx_v, rows_v, sem):
    wid = lax.axis_index("s") * NC + lax.axis_index("c")
    base = wid * TOK_PER_W
    row_base = base % HTOK
    col0 = (base // HTOK) * EMB_DIM

    def step(i, _):
        off = base + i * CHUNK
        pltpu.sync_copy(idx_hbm.at[pl.ds(off, CHUNK)], idx_v)
        copies = []
        for j in range(N_SUB):
            copies.append(
                pltpu.async_copy(
                    table_hbm.at[idx_v.at[pl.ds(j * STREAM, STREAM)]],
                    rows_v.at[pl.ds(j * STREAM, STREAM)],
                    sem,
                )
            )
        for c in copies:
            c.wait()
        pltpu.sync_copy(
            rows_v,
            out_hbm.at[pl.ds(row_base + i * CHUNK, CHUNK), pl.ds(col0, EMB_DIM)],
        )
        return ()

    lax.fori_loop(0, N_STEPS, step, (), unroll=False)


def _sc_gather(table, idx_flat):
    mesh = plsc.VectorSubcoreMesh(core_axis_name="c", subcore_axis_name="s")
    k = functools.partial(
        pl.kernel,
        mesh=mesh,
        out_type=jax.ShapeDtypeStruct((HTOK, 2 * EMB_DIM), jnp.float32),
        scratch_types=[
            pltpu.VMEM((CHUNK,), jnp.int32),
            pltpu.VMEM((CHUNK, EMB_DIM), jnp.float32),
            pltpu.SemaphoreType.DMA,
        ],
        compiler_params=pltpu.CompilerParams(use_tc_tiling_on_sc=False),
    )(_sc_gather_body)
    return k(table, idx_flat)


BLK = 2048
GRID = HTOK // BLK


def _tc_body(xlo, xhi, vlo, vhi, clo, chi, w1t, b1t, w1v, b1v, wcat, gath, out):
    w1tc, b1tc, w1vc, b1vc = w1t[...], b1t[...], w1v[...], b1v[...]
    wc = wcat[...]

    def cve(x, v, cm):
        h_t = jnp.tanh(x * w1tc + b1tc)
        h_v = jnp.tanh(v * w1vc + b1vc) * cm
        h = jnp.concatenate([h_t, h_v], axis=0)
        return lax.dot_general(
            h, wc, (((0,), (0,)), ((), ())),
            preferred_element_type=jnp.float32,
        )

    g = gath[...]
    lo = cve(xlo[0], vlo[0], clo[0]) + g[:, :EMB_DIM]
    hi = cve(xhi[0], vhi[0], chi[0]) + g[:, EMB_DIM:]
    out[...] = jnp.concatenate([lo, hi], axis=1)


def _tc_fused(xt, xv, cmf, w1t, b1t, w1v, b1v, wcat, gath):
    row = lambda i: (i, 0)
    zero = lambda i: (0, 0)
    lo3 = lambda i: (i, 0, 0)
    hi3 = lambda i: (i + GRID, 0, 0)
    xspec = lambda f: pl.BlockSpec((1, 1, BLK), f)
    specs = [
        xspec(lo3), xspec(hi3),
        xspec(lo3), xspec(hi3),
        xspec(lo3), xspec(hi3),
        pl.BlockSpec((HID, 1), zero),
        pl.BlockSpec((HID, 1), zero),
        pl.BlockSpec((HID, 1), zero),
        pl.BlockSpec((HID, 1), zero),
        pl.BlockSpec((2 * HID, EMB_DIM), zero),
        pl.BlockSpec((BLK, 2 * EMB_DIM), row),
    ]
    return pl.pallas_call(
        _tc_body,
        grid=(GRID,),
        in_specs=specs,
        out_specs=pl.BlockSpec((BLK, 2 * EMB_DIM), row),
        out_shape=jax.ShapeDtypeStruct((HTOK, 2 * EMB_DIM), jnp.float32),
    )(xt, xt, xv, xv, cmf, cmf, w1t, b1t, w1v, b1v, wcat, gath)


MRB = 128


def _mask_body(vid, mask):
    mask[...] = jnp.clip(vid[...].astype(jnp.float32), 0.0, 1.0)


def _tc_mask(var_id):
    row = lambda i: (i, 0)
    return pl.pallas_call(
        _mask_body,
        grid=(B // MRB,),
        in_specs=[pl.BlockSpec((MRB, L), row)],
        out_specs=pl.BlockSpec((MRB, L), row),
        out_shape=jax.ShapeDtypeStruct((B, L), jnp.float32),
    )(var_id)


def kernel(time, value, var_id, category_mask, W1_t, b1_t, W2_t, W1_v, b1_v, W2_v, emb_table):
    idx_flat = var_id.reshape(NTOK)
    gath = _sc_gather(emb_table, idx_flat)

    n3 = 2 * GRID
    xt = time.reshape(n3, 1, BLK)
    xv = value.reshape(n3, 1, BLK)
    cmf = category_mask.astype(jnp.float32).reshape(n3, 1, BLK)
    wcat = jnp.concatenate([W2_t, W2_v], axis=0)
    out2 = _tc_fused(
        xt, xv, cmf,
        W1_t.reshape(HID, 1), b1_t.reshape(HID, 1),
        W1_v.reshape(HID, 1), b1_v.reshape(HID, 1),
        wcat, gath,
    )
    sum_emb = jnp.concatenate(
        [out2[:, :EMB_DIM], out2[:, EMB_DIM:]], axis=0
    ).reshape(B, L, EMB_DIM)
    mask = _tc_mask(var_id)
    return sum_emb, mask
